# baseline (speedup 1.0000x reference)
import jax
import jax.numpy as jnp
from jax import lax
from jax.experimental import pallas as pl
from jax.experimental.pallas import tpu as pltpu
from jax.experimental.pallas import tpu_sc as plsc

_NC = 2
_NS = 16
_NW = _NC * _NS
_CH = 128
_G = 4


def _rup(n, m=4096):
    return ((n + m - 1) // m) * m


def _padi(x, n, fill):
    p = n - x.shape[0]
    if p == 0:
        return x
    return jnp.concatenate([x, jnp.full((p,), fill, dtype=x.dtype)])


def _padf(x, n):
    p = n - x.shape[0]
    if p == 0:
        return x
    return jnp.concatenate([x, jnp.zeros((p,) + x.shape[1:], dtype=x.dtype)])


def _mesh():
    return plsc.VectorSubcoreMesh(core_axis_name="c", subcore_axis_name="s")


def _sc_gathers(specs, gb=6):
    dm = specs[0][0].shape[1]
    nspec = len(specs)
    nchts = [idx.shape[0] // _NW // _CH for _, idx in specs]
    gb = min([gb] + [max(n, 1) for n in nchts])

    def body(*refs):
        tables = refs[:nspec]
        idxs = refs[nspec:2 * nspec]
        outs = refs[2 * nspec:3 * nspec]
        scr = refs[3 * nspec:]
        ibufs = list(scr[:gb])
        bufs = list(scr[gb:2 * gb])
        isem, gsem, wsem = scr[2 * gb:2 * gb + 3]
        wid = lax.axis_index("s") * _NC + lax.axis_index("c")

        for table_hbm, idx_hbm, out_hbm, ncht in zip(tables, idxs, outs, nchts):
            ngr, rem = divmod(ncht, gb)
            cbase = wid * ncht

            def fire_idx(j, b):
                pltpu.async_copy(idx_hbm.at[pl.ds((cbase + j) * _CH, _CH)],
                                 ibufs[b], isem)

            def drain_idx(j, b):
                pltpu.make_async_copy(idx_hbm.at[pl.ds((cbase + j) * _CH, _CH)],
                                      ibufs[b], isem).wait()

            def fire_gather(b):
                pltpu.async_copy(table_hbm.at[ibufs[b]], bufs[b], gsem)

            def drain_gather(b):
                pltpu.make_async_copy(table_hbm.at[ibufs[b]], bufs[b],
                                      gsem).wait()

            def fire_wb(j, b):
                pltpu.async_copy(bufs[b],
                                 out_hbm.at[pl.ds((cbase + j) * _CH, _CH)],
                                 wsem)

            def drain_wb(j, b):
                pltpu.make_async_copy(bufs[b],
                                      out_hbm.at[pl.ds((cbase + j) * _CH, _CH)],
                                      wsem).wait()

            def grp(g, carry):
                @pl.when(g > 0)
                def _():
                    for b in range(gb):
                        drain_wb((g - 1) * gb + b, b)
                for b in range(gb):
                    fire_idx(g * gb + b, b)
                for b in range(gb):
                    drain_idx(g * gb + b, b)
                for b in range(gb):
                    fire_gather(b)
                for b in range(gb):
                    drain_gather(b)
                for b in range(gb):
                    fire_wb(g * gb + b, b)
                return carry

            if ngr > 0:
                lax.fori_loop(0, ngr, grp, 0)
            tail = ngr * gb
            if rem > 0:
                if ngr > 0:
                    for b in range(gb):
                        drain_wb(tail - gb + b, b)
                for b in range(rem):
                    fire_idx(tail + b, b)
                for b in range(rem):
                    drain_idx(tail + b, b)
                for b in range(rem):
                    fire_gather(b)
                for b in range(rem):
                    drain_gather(b)
                for b in range(rem):
                    fire_wb(tail + b, b)
                for b in range(rem):
                    drain_wb(tail + b, b)
            elif ngr > 0:
                for b in range(gb):
                    drain_wb(tail - gb + b, b)

    f = pl.kernel(
        body,
        out_type=[jax.ShapeDtypeStruct((idx.shape[0], dm), jnp.float32)
                  for _, idx in specs],
        mesh=_mesh(),
        scratch_types=(
            [pltpu.VMEM((_CH,), jnp.int32)] * gb
            + [pltpu.VMEM((_CH, dm), jnp.float32)] * gb
            + [pltpu.SemaphoreType.DMA] * 3
        ),
    )
    args = [t for t, _ in specs] + [i for _, i in specs]
    return f(*args)


def _sc_gather(table, idx):
    return _sc_gathers([(table, idx)])[0]


def _nt_rpt(n_out):
    nt = max(t for t in range(1, 17)
             if n_out % t == 0 and (n_out // t) % 8 == 0)
    return nt, n_out // nt


def _sc_scatter(rows, idx, n_out):
    npad, dm = rows.shape
    ncht = npad // _NW // _CH
    nchk = ncht * _NW
    nt, rpt = _nt_rpt(n_out)
    zeros = jnp.zeros((n_out, dm), jnp.float32)
    acc_b = (n_out + 8) * dm * 4
    per_tile = (2097151 * 4 - acc_b) // 16 - 4 * _CH * 4 - 4096
    nb = max(1, min(_G, per_tile // (_CH * dm * 4)))
    ngr, rem = divmod(ncht, nb)

    def body(rows_hbm, idx_hbm, z_hbm, out_hbm, *scr):
        ibufs = list(scr[:nb])
        bufs = list(scr[nb:2 * nb])
        lsem, asem, acc = scr[2 * nb], scr[2 * nb + 1], scr[2 * nb + 2]
        c = lax.axis_index("c")
        s_ = lax.axis_index("s")

        @pl.when(s_ < nt)
        def _zero():
            pltpu.sync_copy(z_hbm.at[pl.ds(s_ * rpt, rpt)],
                            acc.at[pl.ds(s_ * rpt, rpt)])

        cbase = c * (nchk // 2) + s_ * ncht
        plsc.subcore_barrier()

        def fire_loads(j, b):
            pltpu.async_copy(idx_hbm.at[pl.ds((cbase + j) * _CH, _CH)],
                             ibufs[b], lsem)
            pltpu.async_copy(rows_hbm.at[pl.ds((cbase + j) * _CH, _CH)],
                             bufs[b], lsem)

        def drain_loads(j, b):
            pltpu.make_async_copy(idx_hbm.at[pl.ds((cbase + j) * _CH, _CH)],
                                  ibufs[b], lsem).wait()
            pltpu.make_async_copy(rows_hbm.at[pl.ds((cbase + j) * _CH, _CH)],
                                  bufs[b], lsem).wait()

        def fire_add(b):
            pltpu.async_copy(bufs[b], acc.at[ibufs[b]], asem, add=True)

        def drain_add(b):
            pltpu.make_async_copy(bufs[b], acc.at[ibufs[b]], asem).wait()

        def grp(g, carry):
            @pl.when(g > 0)
            def _():
                for b in range(nb):
                    drain_add(b)
            for b in range(nb):
                fire_loads(g * nb + b, b)
            for b in range(nb):
                drain_loads(g * nb + b, b)
            for b in range(nb):
                fire_add(b)
            return carry

        if ngr > 0:
            lax.fori_loop(0, ngr, grp, 0)
        tail = ngr * nb
        if rem > 0:
            if ngr > 0:
                for b in range(nb):
                    drain_add(b)
            for b in range(rem):
                fire_loads(tail + b, b)
            for b in range(rem):
                drain_loads(tail + b, b)
            for b in range(rem):
                fire_add(b)
            for b in range(rem):
                drain_add(b)
        elif ngr > 0:
            for b in range(nb):
                drain_add(b)

        plsc.subcore_barrier()

        @pl.when(s_ < nt)
        def _out():
            pltpu.sync_copy(acc.at[pl.ds(s_ * rpt, rpt)],
                            out_hbm.at[pl.ds(c * n_out + s_ * rpt, rpt)])

    f = pl.kernel(
        body,
        out_type=jax.ShapeDtypeStruct((2 * n_out, dm), jnp.float32),
        mesh=_mesh(),
        scratch_types=(
            [pltpu.VMEM((_CH,), jnp.int32)] * nb
            + [pltpu.VMEM((_CH, dm), jnp.float32)] * nb
            + [pltpu.SemaphoreType.DMA] * 2
            + [pltpu.VMEM_SHARED((n_out + 8, dm), jnp.float32)]
        ),
    )
    return f(rows, idx, zeros)


def _sc_pool(table, src, w, dst, n_out):
    npad = src.shape[0]
    dm = table.shape[1]
    ncht = npad // _NW // _CH
    nchk = ncht * _NW
    perw = ncht * _CH
    nt, rpt = _nt_rpt(n_out)
    zeros = jnp.zeros((n_out, dm), jnp.float32)
    npairs, rem1 = divmod(ncht, 2)

    def body(table_hbm, src_hbm, w_hbm, dst_hbm, z_hbm, out_hbm,
             srcA, srcB, dstA, dstB, w_v, bufA, bufB, gsem, asemA, asemB, acc):
        c = lax.axis_index("c")
        s_ = lax.axis_index("s")

        @pl.when(s_ < nt)
        def _zero():
            pltpu.sync_copy(z_hbm.at[pl.ds(s_ * rpt, rpt)],
                            acc.at[pl.ds(s_ * rpt, rpt)])

        cbase = c * (nchk // 2) + s_ * ncht
        pltpu.sync_copy(w_hbm.at[pl.ds(cbase * _CH, perw)], w_v.at[pl.ds(0, perw)])
        plsc.subcore_barrier()

        def load_idx(j, sref, dref):
            pltpu.sync_copy(src_hbm.at[pl.ds((cbase + j) * _CH, _CH)], sref)
            pltpu.sync_copy(dst_hbm.at[pl.ds((cbase + j) * _CH, _CH)], dref)

        def gather(sref, buf):
            pltpu.async_copy(table_hbm.at[sref], buf, gsem).wait()

        def scale(j, buf):
            def row(r, carry):
                q = j * _CH + r
                wv = jnp.full((16,), w_v[pl.ds(q, 16)][0], jnp.float32)
                for k in range(dm // 16):
                    sl = pl.ds(k * 16, 16)
                    buf[r, sl] = buf[r, sl] * wv
                return carry
            lax.fori_loop(0, _CH, row, 0)

        def fire_add(dref, buf, sem):
            pltpu.async_copy(buf, acc.at[dref], sem, add=True)

        def drain_add(dref, buf, sem):
            pltpu.make_async_copy(buf, acc.at[dref], sem).wait()

        def pair(g, carry):
            jA, jB = 2 * g, 2 * g + 1

            @pl.when(g > 0)
            def _():
                drain_add(dstA, bufA, asemA)

            load_idx(jA, srcA, dstA)
            gather(srcA, bufA)
            scale(jA, bufA)
            fire_add(dstA, bufA, asemA)

            @pl.when(g > 0)
            def _():
                drain_add(dstB, bufB, asemB)

            load_idx(jB, srcB, dstB)
            gather(srcB, bufB)
            scale(jB, bufB)
            fire_add(dstB, bufB, asemB)
            return carry

        if npairs > 0:
            lax.fori_loop(0, npairs, pair, 0)
            drain_add(dstA, bufA, asemA)
        if rem1:
            jr = 2 * npairs
            load_idx(jr, srcA, dstA)
            gather(srcA, bufA)
            scale(jr, bufA)
            if npairs > 0:
                drain_add(dstB, bufB, asemB)
            fire_add(dstA, bufA, asemA)
            drain_add(dstA, bufA, asemA)
        elif npairs > 0:
            drain_add(dstB, bufB, asemB)

        plsc.subcore_barrier()

        @pl.when(s_ < nt)
        def _out():
            pltpu.sync_copy(acc.at[pl.ds(s_ * rpt, rpt)],
                            out_hbm.at[pl.ds(c * n_out + s_ * rpt, rpt)])

    f = pl.kernel(
        body,
        out_type=jax.ShapeDtypeStruct((2 * n_out, dm), jnp.float32),
        mesh=_mesh(),
        scratch_types=(
            [pltpu.VMEM((_CH,), jnp.int32)] * 4
            + [pltpu.VMEM((perw + 16,), jnp.float32)]
            + [pltpu.VMEM((_CH, dm), jnp.float32)] * 2
            + [pltpu.SemaphoreType.DMA] * 3
            + [pltpu.VMEM_SHARED((n_out + 8, dm), jnp.float32)]
        ),
    )
    return f(table, src, w, dst, zeros)


def _pick_bn(n, cap=2048):
    bn = 8
    for d in range(8, cap + 1, 8):
        if n % d == 0:
            bn = d
    return bn


def _pack_bf(v):
    d = v.shape[1]
    u = lax.bitcast_convert_type(v, jnp.int32)
    r = u + 0x7FFF + jnp.bitwise_and(lax.shift_right_logical(u, 16), 1)
    b = lax.shift_right_logical(r, 16)
    lo = jnp.bitwise_and(b[:, :d // 2], 0xFFFF)
    hi = lax.shift_left(b[:, d // 2:], 16)
    return jnp.bitwise_or(lo, hi)


def _unpack_bf(v):
    lo = lax.bitcast_convert_type(lax.shift_left(v, 16), jnp.float32)
    hi = lax.bitcast_convert_type(jnp.bitwise_and(v, jnp.int32(-65536)),
                                  jnp.float32)
    return jnp.concatenate([lo, hi], axis=1)


def _l1n(x):
    return x / jnp.clip(jnp.sum(jnp.abs(x), axis=-1, keepdims=True), 1e-12, None)


def _l1_call(x):
    n, dm = x.shape
    bn = _pick_bn(n)

    def body(x_ref, o_ref, bf_ref):
        v = x_ref[...]
        o_ref[...] = _l1n(v)
        bf_ref[...] = _pack_bf(v)

    return pl.pallas_call(
        body,
        grid=(n // bn,),
        in_specs=[pl.BlockSpec((bn, dm), lambda i: (i, 0))],
        out_specs=[pl.BlockSpec((bn, dm), lambda i: (i, 0)),
                   pl.BlockSpec((bn, dm // 2), lambda i: (i, 0))],
        out_shape=[jax.ShapeDtypeStruct((n, dm), jnp.float32),
                   jax.ShapeDtypeStruct((n, dm // 2), jnp.int32)],
    )(x)


def _mlp2(n, groups, w1s, b1, w2, b2, res=None, out_gelu=False,
          l1_extra=False, scale_w=None, bf_extra=False):
    h = w2.shape[0]
    dout = w2.shape[1]
    bn = _pick_bn(n)
    grid = n // bn

    ins, specs = [], []
    for grp in groups:
        for arr, off in grp:
            d = arr.shape[1]
            ob = off // bn
            ins.append(arr)
            specs.append(pl.BlockSpec((bn, d), lambda i, ob=ob: (i + ob, 0)))
    for w in w1s:
        ins.append(w)
        specs.append(pl.BlockSpec(w.shape, lambda i: (0, 0)))
    ins += [b1.reshape(1, h), w2, b2.reshape(1, dout)]
    specs += [pl.BlockSpec((1, h), lambda i: (0, 0)),
              pl.BlockSpec(w2.shape, lambda i: (0, 0)),
              pl.BlockSpec((1, dout), lambda i: (0, 0))]
    if res is not None:
        ins.append(res)
        specs.append(pl.BlockSpec((bn, dout), lambda i: (i, 0)))
    if scale_w is not None:
        ins.append(scale_w)
        specs.append(pl.BlockSpec((bn, 1), lambda i: (i, 0)))

    nout = 1 + int(l1_extra) + int(scale_w is not None) + int(bf_extra)
    gsizes = [len(g) for g in groups]
    ng = len(groups)

    def body(*refs):
        it = iter(refs)
        xs = [[next(it) for _ in range(gsizes[g])] for g in range(ng)]
        ws = [next(it) for _ in range(ng)]
        b1r, w2r, b2r = next(it), next(it), next(it)
        resr = next(it) if res is not None else None
        swr = next(it) if scale_w is not None else None
        outs = [next(it) for _ in range(nout)]
        def _ld(ref):
            v = ref[...]
            if v.dtype == jnp.int32:
                v = _unpack_bf(v)
            return v.astype(jnp.float32)

        acc = None
        for grp_refs, wref in zip(xs, ws):
            x = _ld(grp_refs[0])
            for r2 in grp_refs[1:]:
                x = x + _ld(r2)
            d = jnp.dot(x, wref[...], preferred_element_type=jnp.float32)
            acc = d if acc is None else acc + d
        hh = jax.nn.gelu(acc + b1r[...])
        o = jnp.dot(hh, w2r[...], preferred_element_type=jnp.float32) + b2r[...]
        if out_gelu:
            o = jax.nn.gelu(o)
        if resr is not None:
            o = o + resr[...]
        outs[0][...] = o
        k = 1
        if l1_extra:
            outs[k][...] = _l1n(o)
            k += 1
        if swr is not None:
            outs[k][...] = o * swr[...]
            k += 1
        if bf_extra:
            outs[k][...] = _pack_bf(o)

    out_shape = [jax.ShapeDtypeStruct((n, dout), jnp.float32)] * (
        nout - int(bf_extra))
    out_shape += [jax.ShapeDtypeStruct((n, dout // 2), jnp.int32)] * int(bf_extra)
    out_specs = [pl.BlockSpec((bn, dout), lambda i: (i, 0))] * (
        nout - int(bf_extra))
    out_specs += [pl.BlockSpec((bn, dout // 2), lambda i: (i, 0))] * int(bf_extra)
    outs = pl.pallas_call(
        body,
        grid=(grid,),
        in_specs=specs,
        out_specs=out_specs,
        out_shape=out_shape,
    )(*ins)
    return outs[0] if nout == 1 else outs


def _split_w(w, dims):
    parts, o = [], 0
    for d in dims:
        parts.append(w[o:o + d])
        o += d
    return parts


def kernel(nodes, edges, semb, graph, bgraph, bweights, sgraph, sweights, params):
    nn, dm = nodes.shape
    nsu = semb.shape[0]
    ne = graph.shape[1]
    nb = bgraph.shape[1]
    ns = sgraph.shape[1]
    nep, nbp, nsp = _rup(ne), _rup(nb), _rup(ns)

    g0 = _padi(graph[0], nep, 0)
    g1 = _padi(graph[1], nep, 0)
    g1s = _padi(graph[1], nep, nn)
    bg0 = _padi(bgraph[0], nbp, 0)
    bg1 = _padi(bgraph[1], nbp, 0)
    bg0s = _padi(bgraph[0], nbp, nn)
    bg1s = _padi(bgraph[1], nbp, nsu)
    sg0 = _padi(sgraph[0], nsp, 0)
    sg1 = _padi(sgraph[1], nsp, 0)
    sg1s = _padi(sgraph[1], nsp, nsu)
    bw = _padf(bweights[:, 0], nbp)
    sw = _padf(sweights, nsp)
    edges_p = _padf(edges, nep)

    p = params

    nl1 = _l1_call(nodes)[0]
    pool = _sc_pool(nl1, bg0, bw, bg1s, nsu)

    (w1, b1), (w2, b2) = p['snode_enc']
    w1a, w1b = _split_w(w1, [semb.shape[1], dm])
    snodes = _mlp2(nsu, [[(semb, 0)], [(pool, 0), (pool, nsu)]],
                   [w1a, w1b], b1, w2, b2, out_gelu=True)

    (w1, b1), (w2, b2) = p['sedge_enc']
    w1a, w1b = _split_w(w1, [dm, dm])
    sg0r, sg1r = _sc_gathers([(snodes, sg0), (snodes, sg1)])
    sedges = _mlp2(nsp, [[(sg0r, 0)], [(sg1r, 0)]],
                   [w1a, w1b], b1, w2, b2, out_gelu=True)

    for cell in p['cells']:
        n0r, n1r = _sc_gathers([(nodes, g0), (nodes, g1)])
        sg0r, sg1r = _sc_gathers([(snodes, sg0), (snodes, sg1)])
        down = _sc_pool(snodes, bg1, bw, bg0s, nn)

        (w1, b1), (w2, b2) = cell['edge']
        wa, wb, wc = _split_w(w1, [dm, dm, dm])
        edges_p = _mlp2(nep, [[(n0r, 0)], [(n1r, 0)], [(edges_p, 0)]],
                        [wa, wb, wc], b1, w2, b2, res=edges_p)

        (w1, b1), (w2, b2) = cell['sedge']
        wa, wb, wc = _split_w(w1, [dm, dm, dm])
        sedges, sedges_w = _mlp2(nsp, [[(sg0r, 0)], [(sg1r, 0)], [(sedges, 0)]],
                                 [wa, wb, wc], b1, w2, b2, res=sedges,
                                 scale_w=sw)

        sagg = _sc_scatter(sedges_w, sg1s, nsu)
        eagg = _sc_scatter(edges_p, g1s, nn)

        (w1, b1), (w2, b2) = cell['node']
        wa, wb, wc = _split_w(w1, [dm, dm, dm])
        nodes, nl1 = _mlp2(nn, [[(nodes, 0)],
                                [(eagg, 0), (eagg, nn)],
                                [(down, 0), (down, nn)]],
                           [wa, wb, wc], b1, w2, b2, res=nodes, l1_extra=True)

        up = _sc_pool(nl1, bg0, bw, bg1s, nsu)

        (w1, b1), (w2, b2) = cell['snode']
        wa, wb, wc = _split_w(w1, [dm, dm, dm])
        snodes = _mlp2(nsu, [[(snodes, 0)],
                             [(sagg, 0), (sagg, nsu)],
                             [(up, 0), (up, nsu)]],
                       [wa, wb, wc], b1, w2, b2, res=snodes)

    (w1, b1), (w2, b2) = p['out_clf']
    w1a, w1b = _split_w(w1, [dm, dm])
    fn, fs = _sc_gathers([(nodes, bg0), (snodes, bg1)])
    logits = _mlp2(nbp, [[(fn, 0)], [(fs, 0)]],
                   [w1a, w1b], b1, w2, b2)
    return logits[:nb, 0]


# speedup vs baseline: 1.7387x; 1.7387x over previous
import jax
import jax.numpy as jnp
from jax import lax
from jax.experimental import pallas as pl
from jax.experimental.pallas import tpu as pltpu
from jax.experimental.pallas import tpu_sc as plsc

_NC = 2
_NS = 16
_NW = _NC * _NS
_CH = 128
_G = 4


def _rup(n, m=4096):
    return ((n + m - 1) // m) * m


def _padi(x, n, fill):
    p = n - x.shape[0]
    if p == 0:
        return x
    return jnp.concatenate([x, jnp.full((p,), fill, dtype=x.dtype)])


def _padf(x, n):
    p = n - x.shape[0]
    if p == 0:
        return x
    return jnp.concatenate([x, jnp.zeros((p,) + x.shape[1:], dtype=x.dtype)])


def _mesh():
    return plsc.VectorSubcoreMesh(core_axis_name="c", subcore_axis_name="s")


def _sc_gathers(specs, gb=6):
    dm = specs[0][0].shape[1]
    nspec = len(specs)
    nchts = [idx.shape[0] // _NW // _CH for _, idx in specs]
    utabs, tslot = [], []
    for t, _ in specs:
        for k, u in enumerate(utabs):
            if u is t:
                tslot.append(k)
                break
        else:
            tslot.append(len(utabs))
            utabs.append(t)
    uvs = [t.shape[0] for t in utabs]
    vtot = sum(_rup(v, 8) for v in uvs)
    uoffs = []
    o = 0
    for v in uvs:
        uoffs.append(o)
        o += _rup(v, 8)
    voffs = [uoffs[k] for k in tslot]
    per_tile = (2097151 * 4 - vtot * dm * 4) // 16 - 6 * _CH * 4 - 4096
    gb = max(1, min(gb, per_tile // (_CH * dm * 4)))

    def body(*refs):
        tables = refs[:len(utabs)]
        idxs = refs[len(utabs):len(utabs) + nspec]
        outs = refs[len(utabs) + nspec:len(utabs) + 2 * nspec]
        scr = refs[len(utabs) + 2 * nspec:]
        ibufs = list(scr[:gb])
        bufs = list(scr[gb:2 * gb])
        isem, gsem, wsem, tab = scr[2 * gb:2 * gb + 4]
        c = lax.axis_index("c")
        s_ = lax.axis_index("s")
        wid = s_ * _NC + c

        for t_hbm, v, vo in zip(tables, uvs, uoffs):
            ntz, rpz = _nt_rpt(v)
            @pl.when(s_ < ntz)
            def _stage():
                pltpu.sync_copy(t_hbm.at[pl.ds(s_ * rpz, rpz)],
                                tab.at[pl.ds(vo + s_ * rpz, rpz)])
        plsc.subcore_barrier()

        for spec_i, (idx_hbm, out_hbm, ncht) in enumerate(zip(idxs, outs, nchts)):
            ngr, rem = divmod(ncht, gb)
            cbase = wid * ncht
            vo = voffs[spec_i]

            def fire_idx(j, b):
                pltpu.async_copy(idx_hbm.at[pl.ds((cbase + j) * _CH, _CH)],
                                 ibufs[b], isem)

            def drain_idx(j, b):
                pltpu.make_async_copy(idx_hbm.at[pl.ds((cbase + j) * _CH, _CH)],
                                      ibufs[b], isem).wait()

            def bias(b):
                if vo:
                    ibufs[b][...] = ibufs[b][...] + vo

            def fire_gather(b):
                pltpu.async_copy(tab.at[ibufs[b]], bufs[b], gsem)

            def drain_gather(b):
                pltpu.make_async_copy(tab.at[ibufs[b]], bufs[b], gsem).wait()

            def fire_wb(j, b):
                pltpu.async_copy(bufs[b],
                                 out_hbm.at[pl.ds((cbase + j) * _CH, _CH)],
                                 wsem)

            def drain_wb(j, b):
                pltpu.make_async_copy(bufs[b],
                                      out_hbm.at[pl.ds((cbase + j) * _CH, _CH)],
                                      wsem).wait()

            def grp(g, carry):
                @pl.when(g > 0)
                def _():
                    for b in range(gb):
                        drain_wb((g - 1) * gb + b, b)
                for b in range(gb):
                    fire_idx(g * gb + b, b)
                for b in range(gb):
                    drain_idx(g * gb + b, b)
                    bias(b)
                for b in range(gb):
                    fire_gather(b)
                for b in range(gb):
                    drain_gather(b)
                for b in range(gb):
                    fire_wb(g * gb + b, b)
                return carry

            if ngr > 0:
                lax.fori_loop(0, ngr, grp, 0)
            tail = ngr * gb
            if rem > 0:
                if ngr > 0:
                    for b in range(gb):
                        drain_wb(tail - gb + b, b)
                for b in range(rem):
                    fire_idx(tail + b, b)
                for b in range(rem):
                    drain_idx(tail + b, b)
                    bias(b)
                for b in range(rem):
                    fire_gather(b)
                for b in range(rem):
                    drain_gather(b)
                for b in range(rem):
                    fire_wb(tail + b, b)
                for b in range(rem):
                    drain_wb(tail + b, b)
            elif ngr > 0:
                for b in range(gb):
                    drain_wb(tail - gb + b, b)

    f = pl.kernel(
        body,
        out_type=[jax.ShapeDtypeStruct((idx.shape[0], dm), jnp.float32)
                  for _, idx in specs],
        mesh=_mesh(),
        scratch_types=(
            [pltpu.VMEM((_CH,), jnp.int32)] * gb
            + [pltpu.VMEM((_CH, dm), jnp.float32)] * gb
            + [pltpu.SemaphoreType.DMA] * 3
            + [pltpu.VMEM_SHARED((vtot, dm), jnp.float32)]
        ),
    )
    args = utabs + [i for _, i in specs]
    return f(*args)


def _sc_gather(table, idx):
    return _sc_gathers([(table, idx)])[0]


def _nt_rpt(n_out):
    nt = max(t for t in range(1, 17)
             if n_out % t == 0 and (n_out // t) % 8 == 0)
    return nt, n_out // nt


def _sc_scatter(rows, idx, n_out):
    npad, dm = rows.shape
    ncht = npad // _NW // _CH
    nchk = ncht * _NW
    nt, rpt = _nt_rpt(n_out)
    zeros = jnp.zeros((n_out, dm), jnp.float32)
    acc_b = (n_out + 8) * dm * 4
    per_tile = (2097151 * 4 - acc_b) // 16 - 4 * _CH * 4 - 4096
    nb = max(1, min(_G, per_tile // (_CH * dm * 4)))
    ngr, rem = divmod(ncht, nb)

    def body(rows_hbm, idx_hbm, z_hbm, out_hbm, *scr):
        ibufs = list(scr[:nb])
        bufs = list(scr[nb:2 * nb])
        lsem, asem, acc = scr[2 * nb], scr[2 * nb + 1], scr[2 * nb + 2]
        c = lax.axis_index("c")
        s_ = lax.axis_index("s")

        @pl.when(s_ < nt)
        def _zero():
            pltpu.sync_copy(z_hbm.at[pl.ds(s_ * rpt, rpt)],
                            acc.at[pl.ds(s_ * rpt, rpt)])

        cbase = c * (nchk // 2) + s_ * ncht
        plsc.subcore_barrier()

        def fire_loads(j, b):
            pltpu.async_copy(idx_hbm.at[pl.ds((cbase + j) * _CH, _CH)],
                             ibufs[b], lsem)
            pltpu.async_copy(rows_hbm.at[pl.ds((cbase + j) * _CH, _CH)],
                             bufs[b], lsem)

        def drain_loads(j, b):
            pltpu.make_async_copy(idx_hbm.at[pl.ds((cbase + j) * _CH, _CH)],
                                  ibufs[b], lsem).wait()
            pltpu.make_async_copy(rows_hbm.at[pl.ds((cbase + j) * _CH, _CH)],
                                  bufs[b], lsem).wait()

        def fire_add(b):
            pltpu.async_copy(bufs[b], acc.at[ibufs[b]], asem, add=True)

        def drain_add(b):
            pltpu.make_async_copy(bufs[b], acc.at[ibufs[b]], asem).wait()

        def grp(g, carry):
            @pl.when(g > 0)
            def _():
                for b in range(nb):
                    drain_add(b)
            for b in range(nb):
                fire_loads(g * nb + b, b)
            for b in range(nb):
                drain_loads(g * nb + b, b)
            for b in range(nb):
                fire_add(b)
            return carry

        if ngr > 0:
            lax.fori_loop(0, ngr, grp, 0)
        tail = ngr * nb
        if rem > 0:
            if ngr > 0:
                for b in range(nb):
                    drain_add(b)
            for b in range(rem):
                fire_loads(tail + b, b)
            for b in range(rem):
                drain_loads(tail + b, b)
            for b in range(rem):
                fire_add(b)
            for b in range(rem):
                drain_add(b)
        elif ngr > 0:
            for b in range(nb):
                drain_add(b)

        plsc.subcore_barrier()

        @pl.when(s_ < nt)
        def _out():
            pltpu.sync_copy(acc.at[pl.ds(s_ * rpt, rpt)],
                            out_hbm.at[pl.ds(c * n_out + s_ * rpt, rpt)])

    f = pl.kernel(
        body,
        out_type=jax.ShapeDtypeStruct((2 * n_out, dm), jnp.float32),
        mesh=_mesh(),
        scratch_types=(
            [pltpu.VMEM((_CH,), jnp.int32)] * nb
            + [pltpu.VMEM((_CH, dm), jnp.float32)] * nb
            + [pltpu.SemaphoreType.DMA] * 2
            + [pltpu.VMEM_SHARED((n_out + 8, dm), jnp.float32)]
        ),
    )
    return f(rows, idx, zeros)


def _sc_pool(table, src, w, dst, n_out):
    npad = src.shape[0]
    dm = table.shape[1]
    ncht = npad // _NW // _CH
    nchk = ncht * _NW
    perw = ncht * _CH
    nt, rpt = _nt_rpt(n_out)
    zeros = jnp.zeros((n_out, dm), jnp.float32)
    npairs, rem1 = divmod(ncht, 2)

    def body(table_hbm, src_hbm, w_hbm, dst_hbm, z_hbm, out_hbm,
             srcA, srcB, dstA, dstB, w_v, bufA, bufB, gsem, asemA, asemB, acc):
        c = lax.axis_index("c")
        s_ = lax.axis_index("s")

        @pl.when(s_ < nt)
        def _zero():
            pltpu.sync_copy(z_hbm.at[pl.ds(s_ * rpt, rpt)],
                            acc.at[pl.ds(s_ * rpt, rpt)])

        cbase = c * (nchk // 2) + s_ * ncht
        pltpu.sync_copy(w_hbm.at[pl.ds(cbase * _CH, perw)], w_v.at[pl.ds(0, perw)])
        plsc.subcore_barrier()

        def load_idx(j, sref, dref):
            pltpu.sync_copy(src_hbm.at[pl.ds((cbase + j) * _CH, _CH)], sref)
            pltpu.sync_copy(dst_hbm.at[pl.ds((cbase + j) * _CH, _CH)], dref)

        def gather(sref, buf):
            pltpu.async_copy(table_hbm.at[sref], buf, gsem).wait()

        def scale(j, buf):
            def row(r, carry):
                q = j * _CH + r
                wv = jnp.full((16,), w_v[pl.ds(q, 16)][0], jnp.float32)
                for k in range(dm // 16):
                    sl = pl.ds(k * 16, 16)
                    buf[r, sl] = buf[r, sl] * wv
                return carry
            lax.fori_loop(0, _CH, row, 0)

        def fire_add(dref, buf, sem):
            pltpu.async_copy(buf, acc.at[dref], sem, add=True)

        def drain_add(dref, buf, sem):
            pltpu.make_async_copy(buf, acc.at[dref], sem).wait()

        def pair(g, carry):
            jA, jB = 2 * g, 2 * g + 1

            @pl.when(g > 0)
            def _():
                drain_add(dstA, bufA, asemA)

            load_idx(jA, srcA, dstA)
            gather(srcA, bufA)
            scale(jA, bufA)
            fire_add(dstA, bufA, asemA)

            @pl.when(g > 0)
            def _():
                drain_add(dstB, bufB, asemB)

            load_idx(jB, srcB, dstB)
            gather(srcB, bufB)
            scale(jB, bufB)
            fire_add(dstB, bufB, asemB)
            return carry

        if npairs > 0:
            lax.fori_loop(0, npairs, pair, 0)
            drain_add(dstA, bufA, asemA)
        if rem1:
            jr = 2 * npairs
            load_idx(jr, srcA, dstA)
            gather(srcA, bufA)
            scale(jr, bufA)
            if npairs > 0:
                drain_add(dstB, bufB, asemB)
            fire_add(dstA, bufA, asemA)
            drain_add(dstA, bufA, asemA)
        elif npairs > 0:
            drain_add(dstB, bufB, asemB)

        plsc.subcore_barrier()

        @pl.when(s_ < nt)
        def _out():
            pltpu.sync_copy(acc.at[pl.ds(s_ * rpt, rpt)],
                            out_hbm.at[pl.ds(c * n_out + s_ * rpt, rpt)])

    f = pl.kernel(
        body,
        out_type=jax.ShapeDtypeStruct((2 * n_out, dm), jnp.float32),
        mesh=_mesh(),
        scratch_types=(
            [pltpu.VMEM((_CH,), jnp.int32)] * 4
            + [pltpu.VMEM((perw + 16,), jnp.float32)]
            + [pltpu.VMEM((_CH, dm), jnp.float32)] * 2
            + [pltpu.SemaphoreType.DMA] * 3
            + [pltpu.VMEM_SHARED((n_out + 8, dm), jnp.float32)]
        ),
    )
    return f(table, src, w, dst, zeros)


def _pick_bn(n, cap=2048):
    bn = 8
    for d in range(8, cap + 1, 8):
        if n % d == 0:
            bn = d
    return bn


def _pack_bf(v):
    d = v.shape[1]
    u = lax.bitcast_convert_type(v, jnp.int32)
    r = u + 0x7FFF + jnp.bitwise_and(lax.shift_right_logical(u, 16), 1)
    b = lax.shift_right_logical(r, 16)
    lo = jnp.bitwise_and(b[:, :d // 2], 0xFFFF)
    hi = lax.shift_left(b[:, d // 2:], 16)
    return jnp.bitwise_or(lo, hi)


def _unpack_bf(v):
    lo = lax.bitcast_convert_type(lax.shift_left(v, 16), jnp.float32)
    hi = lax.bitcast_convert_type(jnp.bitwise_and(v, jnp.int32(-65536)),
                                  jnp.float32)
    return jnp.concatenate([lo, hi], axis=1)


def _l1n(x):
    return x / jnp.clip(jnp.sum(jnp.abs(x), axis=-1, keepdims=True), 1e-12, None)


def _l1_call(x):
    n, dm = x.shape
    bn = _pick_bn(n)

    def body(x_ref, o_ref, bf_ref):
        v = x_ref[...]
        o_ref[...] = _l1n(v)
        bf_ref[...] = _pack_bf(v)

    return pl.pallas_call(
        body,
        grid=(n // bn,),
        in_specs=[pl.BlockSpec((bn, dm), lambda i: (i, 0))],
        out_specs=[pl.BlockSpec((bn, dm), lambda i: (i, 0)),
                   pl.BlockSpec((bn, dm // 2), lambda i: (i, 0))],
        out_shape=[jax.ShapeDtypeStruct((n, dm), jnp.float32),
                   jax.ShapeDtypeStruct((n, dm // 2), jnp.int32)],
    )(x)


def _mlp2(n, groups, w1s, b1, w2, b2, res=None, out_gelu=False,
          l1_extra=False, scale_w=None, bf_extra=False):
    h = w2.shape[0]
    dout = w2.shape[1]
    bn = _pick_bn(n)
    grid = n // bn

    ins, specs = [], []
    for grp in groups:
        for arr, off in grp:
            d = arr.shape[1]
            ob = off // bn
            ins.append(arr)
            specs.append(pl.BlockSpec((bn, d), lambda i, ob=ob: (i + ob, 0)))
    for w in w1s:
        ins.append(w)
        specs.append(pl.BlockSpec(w.shape, lambda i: (0, 0)))
    ins += [b1.reshape(1, h), w2, b2.reshape(1, dout)]
    specs += [pl.BlockSpec((1, h), lambda i: (0, 0)),
              pl.BlockSpec(w2.shape, lambda i: (0, 0)),
              pl.BlockSpec((1, dout), lambda i: (0, 0))]
    if res is not None:
        ins.append(res)
        specs.append(pl.BlockSpec((bn, dout), lambda i: (i, 0)))
    if scale_w is not None:
        ins.append(scale_w)
        specs.append(pl.BlockSpec((bn, 1), lambda i: (i, 0)))

    nout = 1 + int(l1_extra) + int(scale_w is not None) + int(bf_extra)
    gsizes = [len(g) for g in groups]
    ng = len(groups)

    def body(*refs):
        it = iter(refs)
        xs = [[next(it) for _ in range(gsizes[g])] for g in range(ng)]
        ws = [next(it) for _ in range(ng)]
        b1r, w2r, b2r = next(it), next(it), next(it)
        resr = next(it) if res is not None else None
        swr = next(it) if scale_w is not None else None
        outs = [next(it) for _ in range(nout)]
        def _ld(ref):
            v = ref[...]
            if v.dtype == jnp.int32:
                v = _unpack_bf(v)
            return v.astype(jnp.float32)

        acc = None
        for grp_refs, wref in zip(xs, ws):
            x = _ld(grp_refs[0])
            for r2 in grp_refs[1:]:
                x = x + _ld(r2)
            d = jnp.dot(x, wref[...], preferred_element_type=jnp.float32)
            acc = d if acc is None else acc + d
        hh = jax.nn.gelu(acc + b1r[...])
        o = jnp.dot(hh, w2r[...], preferred_element_type=jnp.float32) + b2r[...]
        if out_gelu:
            o = jax.nn.gelu(o)
        if resr is not None:
            o = o + resr[...]
        outs[0][...] = o
        k = 1
        if l1_extra:
            outs[k][...] = _l1n(o)
            k += 1
        if swr is not None:
            outs[k][...] = o * swr[...]
            k += 1
        if bf_extra:
            outs[k][...] = _pack_bf(o)

    out_shape = [jax.ShapeDtypeStruct((n, dout), jnp.float32)] * (
        nout - int(bf_extra))
    out_shape += [jax.ShapeDtypeStruct((n, dout // 2), jnp.int32)] * int(bf_extra)
    out_specs = [pl.BlockSpec((bn, dout), lambda i: (i, 0))] * (
        nout - int(bf_extra))
    out_specs += [pl.BlockSpec((bn, dout // 2), lambda i: (i, 0))] * int(bf_extra)
    outs = pl.pallas_call(
        body,
        grid=(grid,),
        in_specs=specs,
        out_specs=out_specs,
        out_shape=out_shape,
    )(*ins)
    return outs[0] if nout == 1 else outs


def _split_w(w, dims):
    parts, o = [], 0
    for d in dims:
        parts.append(w[o:o + d])
        o += d
    return parts


def kernel(nodes, edges, semb, graph, bgraph, bweights, sgraph, sweights, params):
    nn, dm = nodes.shape
    nsu = semb.shape[0]
    ne = graph.shape[1]
    nb = bgraph.shape[1]
    ns = sgraph.shape[1]
    nep, nbp, nsp = _rup(ne), _rup(nb), _rup(ns)

    g0 = _padi(graph[0], nep, 0)
    g1 = _padi(graph[1], nep, 0)
    g1s = _padi(graph[1], nep, nn)
    bg0 = _padi(bgraph[0], nbp, 0)
    bg1 = _padi(bgraph[1], nbp, 0)
    bg0s = _padi(bgraph[0], nbp, nn)
    bg1s = _padi(bgraph[1], nbp, nsu)
    sg0 = _padi(sgraph[0], nsp, 0)
    sg1 = _padi(sgraph[1], nsp, 0)
    sg1s = _padi(sgraph[1], nsp, nsu)
    bw = _padf(bweights[:, 0], nbp)
    sw = _padf(sweights, nsp)
    edges_p = _padf(edges, nep)

    p = params

    nl1 = _l1_call(nodes)[0]
    pool = _sc_pool(nl1, bg0, bw, bg1s, nsu)

    (w1, b1), (w2, b2) = p['snode_enc']
    w1a, w1b = _split_w(w1, [semb.shape[1], dm])
    snodes = _mlp2(nsu, [[(semb, 0)], [(pool, 0), (pool, nsu)]],
                   [w1a, w1b], b1, w2, b2, out_gelu=True)

    (w1, b1), (w2, b2) = p['sedge_enc']
    w1a, w1b = _split_w(w1, [dm, dm])
    sg0r, sg1r = _sc_gathers([(snodes, sg0), (snodes, sg1)])
    sedges = _mlp2(nsp, [[(sg0r, 0)], [(sg1r, 0)]],
                   [w1a, w1b], b1, w2, b2, out_gelu=True)

    for cell in p['cells']:
        n0r, n1r = _sc_gathers([(nodes, g0), (nodes, g1)])
        sg0r, sg1r = _sc_gathers([(snodes, sg0), (snodes, sg1)])
        down = _sc_pool(snodes, bg1, bw, bg0s, nn)

        (w1, b1), (w2, b2) = cell['edge']
        wa, wb, wc = _split_w(w1, [dm, dm, dm])
        edges_p = _mlp2(nep, [[(n0r, 0)], [(n1r, 0)], [(edges_p, 0)]],
                        [wa, wb, wc], b1, w2, b2, res=edges_p)

        (w1, b1), (w2, b2) = cell['sedge']
        wa, wb, wc = _split_w(w1, [dm, dm, dm])
        sedges, sedges_w = _mlp2(nsp, [[(sg0r, 0)], [(sg1r, 0)], [(sedges, 0)]],
                                 [wa, wb, wc], b1, w2, b2, res=sedges,
                                 scale_w=sw)

        sagg = _sc_scatter(sedges_w, sg1s, nsu)
        eagg = _sc_scatter(edges_p, g1s, nn)

        (w1, b1), (w2, b2) = cell['node']
        wa, wb, wc = _split_w(w1, [dm, dm, dm])
        nodes, nl1 = _mlp2(nn, [[(nodes, 0)],
                                [(eagg, 0), (eagg, nn)],
                                [(down, 0), (down, nn)]],
                           [wa, wb, wc], b1, w2, b2, res=nodes, l1_extra=True)

        up = _sc_pool(nl1, bg0, bw, bg1s, nsu)

        (w1, b1), (w2, b2) = cell['snode']
        wa, wb, wc = _split_w(w1, [dm, dm, dm])
        snodes = _mlp2(nsu, [[(snodes, 0)],
                             [(sagg, 0), (sagg, nsu)],
                             [(up, 0), (up, nsu)]],
                       [wa, wb, wc], b1, w2, b2, res=snodes)

    (w1, b1), (w2, b2) = p['out_clf']
    w1a, w1b = _split_w(w1, [dm, dm])
    fn, fs = _sc_gathers([(nodes, bg0), (snodes, bg1)])
    logits = _mlp2(nbp, [[(fn, 0)], [(fs, 0)]],
                   [w1a, w1b], b1, w2, b2)
    return logits[:nb, 0]


# speedup vs baseline: 2.5067x; 1.4418x over previous
import jax
import jax.numpy as jnp
from jax import lax
from jax.experimental import pallas as pl
from jax.experimental.pallas import tpu as pltpu
from jax.experimental.pallas import tpu_sc as plsc

_NC = 2
_NS = 16
_NW = _NC * _NS
_CH = 128
_G = 4


def _rup(n, m=4096):
    return ((n + m - 1) // m) * m


def _padi(x, n, fill):
    p = n - x.shape[0]
    if p == 0:
        return x
    return jnp.concatenate([x, jnp.full((p,), fill, dtype=x.dtype)])


def _padf(x, n):
    p = n - x.shape[0]
    if p == 0:
        return x
    return jnp.concatenate([x, jnp.zeros((p,) + x.shape[1:], dtype=x.dtype)])


def _mesh():
    return plsc.VectorSubcoreMesh(core_axis_name="c", subcore_axis_name="s")


def _sc_gathers(specs, gb=6):
    dm = specs[0][0].shape[1]
    nspec = len(specs)
    nchts = [idx.shape[0] // _NW // _CH for _, idx in specs]
    utabs, tslot = [], []
    for t, _ in specs:
        for k, u in enumerate(utabs):
            if u is t:
                tslot.append(k)
                break
        else:
            tslot.append(len(utabs))
            utabs.append(t)
    uvs = [t.shape[0] for t in utabs]
    vtot = sum(_rup(v, 8) for v in uvs)
    uoffs = []
    o = 0
    for v in uvs:
        uoffs.append(o)
        o += _rup(v, 8)
    voffs = [uoffs[k] for k in tslot]
    per_tile = (2097151 * 4 - vtot * dm * 4) // 16 - 6 * _CH * 4 - 4096
    gb = max(1, min(gb, per_tile // (_CH * dm * 4)))

    def body(*refs):
        tables = refs[:len(utabs)]
        idxs = refs[len(utabs):len(utabs) + nspec]
        outs = refs[len(utabs) + nspec:len(utabs) + 2 * nspec]
        scr = refs[len(utabs) + 2 * nspec:]
        ibufs = list(scr[:gb])
        bufs = list(scr[gb:2 * gb])
        isem, gsem, wsem, tab = scr[2 * gb:2 * gb + 4]
        c = lax.axis_index("c")
        s_ = lax.axis_index("s")
        wid = s_ * _NC + c

        for t_hbm, v, vo in zip(tables, uvs, uoffs):
            ntz, rpz = _nt_rpt(v)
            @pl.when(s_ < ntz)
            def _stage():
                pltpu.sync_copy(t_hbm.at[pl.ds(s_ * rpz, rpz)],
                                tab.at[pl.ds(vo + s_ * rpz, rpz)])
        plsc.subcore_barrier()

        for spec_i, (idx_hbm, out_hbm, ncht) in enumerate(zip(idxs, outs, nchts)):
            ngr, rem = divmod(ncht, gb)
            cbase = wid * ncht
            vo = voffs[spec_i]

            def fire_idx(j, b):
                pltpu.async_copy(idx_hbm.at[pl.ds((cbase + j) * _CH, _CH)],
                                 ibufs[b], isem)

            def drain_idx(j, b):
                pltpu.make_async_copy(idx_hbm.at[pl.ds((cbase + j) * _CH, _CH)],
                                      ibufs[b], isem).wait()

            def bias(b):
                if vo:
                    ibufs[b][...] = ibufs[b][...] + vo

            def fire_gather(b):
                pltpu.async_copy(tab.at[ibufs[b]], bufs[b], gsem)

            def drain_gather(b):
                pltpu.make_async_copy(tab.at[ibufs[b]], bufs[b], gsem).wait()

            def fire_wb(j, b):
                pltpu.async_copy(bufs[b],
                                 out_hbm.at[pl.ds((cbase + j) * _CH, _CH)],
                                 wsem)

            def drain_wb(j, b):
                pltpu.make_async_copy(bufs[b],
                                      out_hbm.at[pl.ds((cbase + j) * _CH, _CH)],
                                      wsem).wait()

            def grp(g, carry):
                @pl.when(g > 0)
                def _():
                    for b in range(gb):
                        drain_wb((g - 1) * gb + b, b)
                for b in range(gb):
                    fire_idx(g * gb + b, b)
                for b in range(gb):
                    drain_idx(g * gb + b, b)
                    bias(b)
                for b in range(gb):
                    fire_gather(b)
                for b in range(gb):
                    drain_gather(b)
                for b in range(gb):
                    fire_wb(g * gb + b, b)
                return carry

            if ngr > 0:
                lax.fori_loop(0, ngr, grp, 0)
            tail = ngr * gb
            if rem > 0:
                if ngr > 0:
                    for b in range(gb):
                        drain_wb(tail - gb + b, b)
                for b in range(rem):
                    fire_idx(tail + b, b)
                for b in range(rem):
                    drain_idx(tail + b, b)
                    bias(b)
                for b in range(rem):
                    fire_gather(b)
                for b in range(rem):
                    drain_gather(b)
                for b in range(rem):
                    fire_wb(tail + b, b)
                for b in range(rem):
                    drain_wb(tail + b, b)
            elif ngr > 0:
                for b in range(gb):
                    drain_wb(tail - gb + b, b)

    f = pl.kernel(
        body,
        out_type=[jax.ShapeDtypeStruct((idx.shape[0], dm), jnp.float32)
                  for _, idx in specs],
        mesh=_mesh(),
        scratch_types=(
            [pltpu.VMEM((_CH,), jnp.int32)] * gb
            + [pltpu.VMEM((_CH, dm), jnp.float32)] * gb
            + [pltpu.SemaphoreType.DMA] * 3
            + [pltpu.VMEM_SHARED((vtot, dm), jnp.float32)]
        ),
    )
    args = utabs + [i for _, i in specs]
    return f(*args)


def _sc_gather(table, idx):
    return _sc_gathers([(table, idx)])[0]


def _nt_rpt(n_out):
    nt = max(t for t in range(1, 17)
             if n_out % t == 0 and (n_out // t) % 8 == 0)
    return nt, n_out // nt


def _sc_scatter(rows, idx, n_out):
    npad, dm = rows.shape
    ncht = npad // _NW // _CH
    nchk = ncht * _NW
    nt, rpt = _nt_rpt(n_out)
    zeros = jnp.zeros((n_out, dm), jnp.float32)
    acc_b = (n_out + 8) * dm * 4
    per_tile = (2097151 * 4 - acc_b) // 16 - 4 * _CH * 4 - 4096
    nb = max(1, min(_G, per_tile // (_CH * dm * 4)))
    ngr, rem = divmod(ncht, nb)

    def body(rows_hbm, idx_hbm, z_hbm, out_hbm, *scr):
        ibufs = list(scr[:nb])
        bufs = list(scr[nb:2 * nb])
        lsem, asem, acc = scr[2 * nb], scr[2 * nb + 1], scr[2 * nb + 2]
        c = lax.axis_index("c")
        s_ = lax.axis_index("s")

        @pl.when(s_ < nt)
        def _zero():
            pltpu.sync_copy(z_hbm.at[pl.ds(s_ * rpt, rpt)],
                            acc.at[pl.ds(s_ * rpt, rpt)])

        cbase = c * (nchk // 2) + s_ * ncht
        plsc.subcore_barrier()

        def fire_loads(j, b):
            pltpu.async_copy(idx_hbm.at[pl.ds((cbase + j) * _CH, _CH)],
                             ibufs[b], lsem)
            pltpu.async_copy(rows_hbm.at[pl.ds((cbase + j) * _CH, _CH)],
                             bufs[b], lsem)

        def drain_loads(j, b):
            pltpu.make_async_copy(idx_hbm.at[pl.ds((cbase + j) * _CH, _CH)],
                                  ibufs[b], lsem).wait()
            pltpu.make_async_copy(rows_hbm.at[pl.ds((cbase + j) * _CH, _CH)],
                                  bufs[b], lsem).wait()

        def fire_add(b):
            pltpu.async_copy(bufs[b], acc.at[ibufs[b]], asem, add=True)

        def drain_add(b):
            pltpu.make_async_copy(bufs[b], acc.at[ibufs[b]], asem).wait()

        def grp(g, carry):
            @pl.when(g > 0)
            def _():
                for b in range(nb):
                    drain_add(b)
            for b in range(nb):
                fire_loads(g * nb + b, b)
            for b in range(nb):
                drain_loads(g * nb + b, b)
            for b in range(nb):
                fire_add(b)
            return carry

        if ngr > 0:
            lax.fori_loop(0, ngr, grp, 0)
        tail = ngr * nb
        if rem > 0:
            if ngr > 0:
                for b in range(nb):
                    drain_add(b)
            for b in range(rem):
                fire_loads(tail + b, b)
            for b in range(rem):
                drain_loads(tail + b, b)
            for b in range(rem):
                fire_add(b)
            for b in range(rem):
                drain_add(b)
        elif ngr > 0:
            for b in range(nb):
                drain_add(b)

        plsc.subcore_barrier()

        @pl.when(s_ < nt)
        def _out():
            pltpu.sync_copy(acc.at[pl.ds(s_ * rpt, rpt)],
                            out_hbm.at[pl.ds(c * n_out + s_ * rpt, rpt)])

    f = pl.kernel(
        body,
        out_type=jax.ShapeDtypeStruct((2 * n_out, dm), jnp.float32),
        mesh=_mesh(),
        scratch_types=(
            [pltpu.VMEM((_CH,), jnp.int32)] * nb
            + [pltpu.VMEM((_CH, dm), jnp.float32)] * nb
            + [pltpu.SemaphoreType.DMA] * 2
            + [pltpu.VMEM_SHARED((n_out + 8, dm), jnp.float32)]
        ),
    )
    return f(rows, idx, zeros)


def _sc_pool(table, src, w, dst, n_out):
    npad = src.shape[0]
    dm = table.shape[1]
    vtab = _rup(table.shape[0], 8)
    ncht = npad // _NW // _CH
    nchk = ncht * _NW
    perw = ncht * _CH
    nt, rpt = _nt_rpt(n_out)
    ntz, rpz = _nt_rpt(table.shape[0])
    zeros = jnp.zeros((n_out, dm), jnp.float32)
    npairs, rem1 = divmod(ncht, 2)

    def body(table_hbm, src_hbm, w_hbm, dst_hbm, z_hbm, out_hbm,
             srcA, srcB, dstA, dstB, w_v, bufA, bufB, gsem, asemA, asemB,
             acc, tab):
        c = lax.axis_index("c")
        s_ = lax.axis_index("s")

        @pl.when(s_ < nt)
        def _zero():
            pltpu.sync_copy(z_hbm.at[pl.ds(s_ * rpt, rpt)],
                            acc.at[pl.ds(s_ * rpt, rpt)])

        @pl.when(s_ < ntz)
        def _stage():
            pltpu.sync_copy(table_hbm.at[pl.ds(s_ * rpz, rpz)],
                            tab.at[pl.ds(s_ * rpz, rpz)])

        cbase = c * (nchk // 2) + s_ * ncht
        pltpu.sync_copy(w_hbm.at[pl.ds(cbase * _CH, perw)], w_v.at[pl.ds(0, perw)])
        plsc.subcore_barrier()

        def load_idx(j, sref, dref):
            pltpu.sync_copy(src_hbm.at[pl.ds((cbase + j) * _CH, _CH)], sref)
            pltpu.sync_copy(dst_hbm.at[pl.ds((cbase + j) * _CH, _CH)], dref)

        def gather(sref, buf):
            pltpu.async_copy(tab.at[sref], buf, gsem).wait()

        def scale(j, buf):
            def row(r, carry):
                q = j * _CH + r
                wv = jnp.full((16,), w_v[pl.ds(q, 16)][0], jnp.float32)
                for k in range(dm // 16):
                    sl = pl.ds(k * 16, 16)
                    buf[r, sl] = buf[r, sl] * wv
                return carry
            lax.fori_loop(0, _CH, row, 0)

        def fire_add(dref, buf, sem):
            pltpu.async_copy(buf, acc.at[dref], sem, add=True)

        def drain_add(dref, buf, sem):
            pltpu.make_async_copy(buf, acc.at[dref], sem).wait()

        def pair(g, carry):
            jA, jB = 2 * g, 2 * g + 1

            @pl.when(g > 0)
            def _():
                drain_add(dstA, bufA, asemA)

            load_idx(jA, srcA, dstA)
            gather(srcA, bufA)
            scale(jA, bufA)
            fire_add(dstA, bufA, asemA)

            @pl.when(g > 0)
            def _():
                drain_add(dstB, bufB, asemB)

            load_idx(jB, srcB, dstB)
            gather(srcB, bufB)
            scale(jB, bufB)
            fire_add(dstB, bufB, asemB)
            return carry

        if npairs > 0:
            lax.fori_loop(0, npairs, pair, 0)
            drain_add(dstA, bufA, asemA)
        if rem1:
            jr = 2 * npairs
            load_idx(jr, srcA, dstA)
            gather(srcA, bufA)
            scale(jr, bufA)
            if npairs > 0:
                drain_add(dstB, bufB, asemB)
            fire_add(dstA, bufA, asemA)
            drain_add(dstA, bufA, asemA)
        elif npairs > 0:
            drain_add(dstB, bufB, asemB)

        plsc.subcore_barrier()

        @pl.when(s_ < nt)
        def _out():
            pltpu.sync_copy(acc.at[pl.ds(s_ * rpt, rpt)],
                            out_hbm.at[pl.ds(c * n_out + s_ * rpt, rpt)])

    f = pl.kernel(
        body,
        out_type=jax.ShapeDtypeStruct((2 * n_out, dm), jnp.float32),
        mesh=_mesh(),
        scratch_types=(
            [pltpu.VMEM((_CH,), jnp.int32)] * 4
            + [pltpu.VMEM((perw + 16,), jnp.float32)]
            + [pltpu.VMEM((_CH, dm), jnp.float32)] * 2
            + [pltpu.SemaphoreType.DMA] * 3
            + [pltpu.VMEM_SHARED((n_out + 8, dm), jnp.float32)]
            + [pltpu.VMEM_SHARED((vtab, dm), jnp.float32)]
        ),
    )
    return f(table, src, w, dst, zeros)


def _pick_bn(n, cap=2048):
    bn = 8
    for d in range(8, cap + 1, 8):
        if n % d == 0:
            bn = d
    return bn


def _pack_bf(v):
    d = v.shape[1]
    u = lax.bitcast_convert_type(v, jnp.int32)
    r = u + 0x7FFF + jnp.bitwise_and(lax.shift_right_logical(u, 16), 1)
    b = lax.shift_right_logical(r, 16)
    lo = jnp.bitwise_and(b[:, :d // 2], 0xFFFF)
    hi = lax.shift_left(b[:, d // 2:], 16)
    return jnp.bitwise_or(lo, hi)


def _unpack_bf(v):
    lo = lax.bitcast_convert_type(lax.shift_left(v, 16), jnp.float32)
    hi = lax.bitcast_convert_type(jnp.bitwise_and(v, jnp.int32(-65536)),
                                  jnp.float32)
    return jnp.concatenate([lo, hi], axis=1)


def _l1n(x):
    return x / jnp.clip(jnp.sum(jnp.abs(x), axis=-1, keepdims=True), 1e-12, None)


def _l1_call(x):
    n, dm = x.shape
    bn = _pick_bn(n)

    def body(x_ref, o_ref, bf_ref):
        v = x_ref[...]
        o_ref[...] = _l1n(v)
        bf_ref[...] = _pack_bf(v)

    return pl.pallas_call(
        body,
        grid=(n // bn,),
        in_specs=[pl.BlockSpec((bn, dm), lambda i: (i, 0))],
        out_specs=[pl.BlockSpec((bn, dm), lambda i: (i, 0)),
                   pl.BlockSpec((bn, dm // 2), lambda i: (i, 0))],
        out_shape=[jax.ShapeDtypeStruct((n, dm), jnp.float32),
                   jax.ShapeDtypeStruct((n, dm // 2), jnp.int32)],
    )(x)


def _mlp2(n, groups, w1s, b1, w2, b2, res=None, out_gelu=False,
          l1_extra=False, scale_w=None, bf_extra=False):
    h = w2.shape[0]
    dout = w2.shape[1]
    bn = _pick_bn(n)
    grid = n // bn

    ins, specs = [], []
    for grp in groups:
        for arr, off in grp:
            d = arr.shape[1]
            ob = off // bn
            ins.append(arr)
            specs.append(pl.BlockSpec((bn, d), lambda i, ob=ob: (i + ob, 0)))
    for w in w1s:
        ins.append(w)
        specs.append(pl.BlockSpec(w.shape, lambda i: (0, 0)))
    ins += [b1.reshape(1, h), w2, b2.reshape(1, dout)]
    specs += [pl.BlockSpec((1, h), lambda i: (0, 0)),
              pl.BlockSpec(w2.shape, lambda i: (0, 0)),
              pl.BlockSpec((1, dout), lambda i: (0, 0))]
    if res is not None:
        ins.append(res)
        specs.append(pl.BlockSpec((bn, dout), lambda i: (i, 0)))
    if scale_w is not None:
        ins.append(scale_w)
        specs.append(pl.BlockSpec((bn, 1), lambda i: (i, 0)))

    nout = 1 + int(l1_extra) + int(scale_w is not None) + int(bf_extra)
    gsizes = [len(g) for g in groups]
    ng = len(groups)

    def body(*refs):
        it = iter(refs)
        xs = [[next(it) for _ in range(gsizes[g])] for g in range(ng)]
        ws = [next(it) for _ in range(ng)]
        b1r, w2r, b2r = next(it), next(it), next(it)
        resr = next(it) if res is not None else None
        swr = next(it) if scale_w is not None else None
        outs = [next(it) for _ in range(nout)]
        def _ld(ref):
            v = ref[...]
            if v.dtype == jnp.int32:
                v = _unpack_bf(v)
            return v.astype(jnp.float32)

        acc = None
        for grp_refs, wref in zip(xs, ws):
            x = _ld(grp_refs[0])
            for r2 in grp_refs[1:]:
                x = x + _ld(r2)
            d = jnp.dot(x, wref[...], preferred_element_type=jnp.float32)
            acc = d if acc is None else acc + d
        hh = jax.nn.gelu(acc + b1r[...])
        o = jnp.dot(hh, w2r[...], preferred_element_type=jnp.float32) + b2r[...]
        if out_gelu:
            o = jax.nn.gelu(o)
        if resr is not None:
            o = o + resr[...]
        outs[0][...] = o
        k = 1
        if l1_extra:
            outs[k][...] = _l1n(o)
            k += 1
        if swr is not None:
            outs[k][...] = o * swr[...]
            k += 1
        if bf_extra:
            outs[k][...] = _pack_bf(o)

    out_shape = [jax.ShapeDtypeStruct((n, dout), jnp.float32)] * (
        nout - int(bf_extra))
    out_shape += [jax.ShapeDtypeStruct((n, dout // 2), jnp.int32)] * int(bf_extra)
    out_specs = [pl.BlockSpec((bn, dout), lambda i: (i, 0))] * (
        nout - int(bf_extra))
    out_specs += [pl.BlockSpec((bn, dout // 2), lambda i: (i, 0))] * int(bf_extra)
    outs = pl.pallas_call(
        body,
        grid=(grid,),
        in_specs=specs,
        out_specs=out_specs,
        out_shape=out_shape,
    )(*ins)
    return outs[0] if nout == 1 else outs


def _split_w(w, dims):
    parts, o = [], 0
    for d in dims:
        parts.append(w[o:o + d])
        o += d
    return parts


def kernel(nodes, edges, semb, graph, bgraph, bweights, sgraph, sweights, params):
    nn, dm = nodes.shape
    nsu = semb.shape[0]
    ne = graph.shape[1]
    nb = bgraph.shape[1]
    ns = sgraph.shape[1]
    nep, nbp, nsp = _rup(ne), _rup(nb), _rup(ns)

    g0 = _padi(graph[0], nep, 0)
    g1 = _padi(graph[1], nep, 0)
    g1s = _padi(graph[1], nep, nn)
    bg0 = _padi(bgraph[0], nbp, 0)
    bg1 = _padi(bgraph[1], nbp, 0)
    bg0s = _padi(bgraph[0], nbp, nn)
    bg1s = _padi(bgraph[1], nbp, nsu)
    sg0 = _padi(sgraph[0], nsp, 0)
    sg1 = _padi(sgraph[1], nsp, 0)
    sg1s = _padi(sgraph[1], nsp, nsu)
    bw = _padf(bweights[:, 0], nbp)
    sw = _padf(sweights, nsp)
    edges_p = _padf(edges, nep)

    p = params

    nl1 = _l1_call(nodes)[0]
    pool = _sc_pool(nl1, bg0, bw, bg1s, nsu)

    (w1, b1), (w2, b2) = p['snode_enc']
    w1a, w1b = _split_w(w1, [semb.shape[1], dm])
    snodes = _mlp2(nsu, [[(semb, 0)], [(pool, 0), (pool, nsu)]],
                   [w1a, w1b], b1, w2, b2, out_gelu=True)

    (w1, b1), (w2, b2) = p['sedge_enc']
    w1a, w1b = _split_w(w1, [dm, dm])
    sg0r, sg1r = _sc_gathers([(snodes, sg0), (snodes, sg1)])
    sedges = _mlp2(nsp, [[(sg0r, 0)], [(sg1r, 0)]],
                   [w1a, w1b], b1, w2, b2, out_gelu=True)

    for cell in p['cells']:
        n0r, n1r = _sc_gathers([(nodes, g0), (nodes, g1)])
        sg0r, sg1r = _sc_gathers([(snodes, sg0), (snodes, sg1)])
        down = _sc_pool(snodes, bg1, bw, bg0s, nn)

        (w1, b1), (w2, b2) = cell['edge']
        wa, wb, wc = _split_w(w1, [dm, dm, dm])
        edges_p = _mlp2(nep, [[(n0r, 0)], [(n1r, 0)], [(edges_p, 0)]],
                        [wa, wb, wc], b1, w2, b2, res=edges_p)

        (w1, b1), (w2, b2) = cell['sedge']
        wa, wb, wc = _split_w(w1, [dm, dm, dm])
        sedges, sedges_w = _mlp2(nsp, [[(sg0r, 0)], [(sg1r, 0)], [(sedges, 0)]],
                                 [wa, wb, wc], b1, w2, b2, res=sedges,
                                 scale_w=sw)

        sagg = _sc_scatter(sedges_w, sg1s, nsu)
        eagg = _sc_scatter(edges_p, g1s, nn)

        (w1, b1), (w2, b2) = cell['node']
        wa, wb, wc = _split_w(w1, [dm, dm, dm])
        nodes, nl1 = _mlp2(nn, [[(nodes, 0)],
                                [(eagg, 0), (eagg, nn)],
                                [(down, 0), (down, nn)]],
                           [wa, wb, wc], b1, w2, b2, res=nodes, l1_extra=True)

        up = _sc_pool(nl1, bg0, bw, bg1s, nsu)

        (w1, b1), (w2, b2) = cell['snode']
        wa, wb, wc = _split_w(w1, [dm, dm, dm])
        snodes = _mlp2(nsu, [[(snodes, 0)],
                             [(sagg, 0), (sagg, nsu)],
                             [(up, 0), (up, nsu)]],
                       [wa, wb, wc], b1, w2, b2, res=snodes)

    (w1, b1), (w2, b2) = p['out_clf']
    w1a, w1b = _split_w(w1, [dm, dm])
    fn, fs = _sc_gathers([(nodes, bg0), (snodes, bg1)])
    logits = _mlp2(nbp, [[(fn, 0)], [(fs, 0)]],
                   [w1a, w1b], b1, w2, b2)
    return logits[:nb, 0]
